# baseline (speedup 1.0000x reference)
import functools

import jax
import jax.numpy as jnp
from jax import lax
from jax.experimental import pallas as pl
from jax.experimental.pallas import tpu as pltpu
from jax.experimental.pallas import tpu_sc as plsc

FEATURE_DIM = 128
DATA_SIZE = 16384
BATCH = 32
T = 0.07
MOMENTUM = 0.5

ROW_BLK = 1024
NUM_BLKS = DATA_SIZE // ROW_BLK
LANES = 16
UNROLL = 4


def _scores_body(fringe_ref, phase_ref, fea_f_ref, fea_p_ref, y_ref,
                 sf_ref, sp_ref, uf_ref, up_ref):
    i = pl.program_id(0)
    fringe = fringe_ref[...]
    phase = phase_ref[...]
    ff = fea_f_ref[...]
    fp = fea_p_ref[...]
    inv_t = jnp.float32(1.0 / T)

    dn_t = (((1,), (1,)), ((), ()))
    sf_ref[...] = lax.dot_general(fp, fringe, dn_t,
                                  preferred_element_type=jnp.float32) * inv_t
    sp_ref[...] = lax.dot_general(ff, phase, dn_t,
                                  preferred_element_type=jnp.float32) * inv_t

    y = y_ref[...]
    r = y - i * ROW_BLK
    col = lax.broadcasted_iota(jnp.int32, (BATCH, ROW_BLK), 1)
    onehot = (col == r).astype(jnp.float32)

    @pl.when(i == 0)
    def _():
        uf_ref[...] = (1.0 - MOMENTUM) * ff
        up_ref[...] = (1.0 - MOMENTUM) * fp

    dn_g = (((1,), (0,)), ((), ()))
    uf_ref[...] += MOMENTUM * lax.dot_general(
        onehot, fringe, dn_g, preferred_element_type=jnp.float32)
    up_ref[...] += MOMENTUM * lax.dot_general(
        onehot, phase, dn_g, preferred_element_type=jnp.float32)

    @pl.when(i == NUM_BLKS - 1)
    def _():
        uf = uf_ref[...]
        up = up_ref[...]
        uf_ref[...] = uf / jnp.maximum(
            jnp.sqrt(jnp.sum(uf * uf, axis=1, keepdims=True)), 1e-30)
        up_ref[...] = up / jnp.maximum(
            jnp.sqrt(jnp.sum(up * up, axis=1, keepdims=True)), 1e-30)


def _scores_call(fringe, phase, ff, fp, yv):
    return pl.pallas_call(
        _scores_body,
        grid=(NUM_BLKS,),
        in_specs=[
            pl.BlockSpec((ROW_BLK, FEATURE_DIM), lambda i: (i, 0)),
            pl.BlockSpec((ROW_BLK, FEATURE_DIM), lambda i: (i, 0)),
            pl.BlockSpec((BATCH, FEATURE_DIM), lambda i: (0, 0)),
            pl.BlockSpec((BATCH, FEATURE_DIM), lambda i: (0, 0)),
            pl.BlockSpec((BATCH, 1), lambda i: (0, 0)),
        ],
        out_specs=[
            pl.BlockSpec((BATCH, ROW_BLK), lambda i: (0, i)),
            pl.BlockSpec((BATCH, ROW_BLK), lambda i: (0, i)),
            pl.BlockSpec((BATCH, FEATURE_DIM), lambda i: (0, 0)),
            pl.BlockSpec((BATCH, FEATURE_DIM), lambda i: (0, 0)),
        ],
        out_shape=[
            jax.ShapeDtypeStruct((BATCH, DATA_SIZE), jnp.float32),
            jax.ShapeDtypeStruct((BATCH, DATA_SIZE), jnp.float32),
            jax.ShapeDtypeStruct((BATCH, FEATURE_DIM), jnp.float32),
            jax.ShapeDtypeStruct((BATCH, FEATURE_DIM), jnp.float32),
        ],
        compiler_params=pltpu.CompilerParams(
            dimension_semantics=("arbitrary",),
        ),
    )(fringe, phase, ff, fp, yv)


def _scatter_body(y_ref, uf_ref, up_ref, fin_ref, pin_ref,
                  fout_ref, pout_ref):
    del y_ref, fin_ref, pin_ref
    fout_ref[...] = uf_ref[...]
    pout_ref[...] = up_ref[...]


def _scatter_call(y, upd_f, upd_p, fringe, phase):
    upd_f3 = upd_f.reshape(BATCH, 1, FEATURE_DIM)
    upd_p3 = upd_p.reshape(BATCH, 1, FEATURE_DIM)
    fringe3 = fringe.reshape(DATA_SIZE, 1, FEATURE_DIM)
    phase3 = phase.reshape(DATA_SIZE, 1, FEATURE_DIM)
    blk = (1, 1, FEATURE_DIM)
    grid_spec = pltpu.PrefetchScalarGridSpec(
        num_scalar_prefetch=1,
        grid=(BATCH,),
        in_specs=[
            pl.BlockSpec(blk, lambda i, y_ref: (i, 0, 0)),
            pl.BlockSpec(blk, lambda i, y_ref: (i, 0, 0)),
            pl.BlockSpec(blk, lambda i, y_ref: (y_ref[i], 0, 0)),
            pl.BlockSpec(blk, lambda i, y_ref: (y_ref[i], 0, 0)),
        ],
        out_specs=[
            pl.BlockSpec(blk, lambda i, y_ref: (y_ref[i], 0, 0)),
            pl.BlockSpec(blk, lambda i, y_ref: (y_ref[i], 0, 0)),
        ],
    )
    nf3, np3 = pl.pallas_call(
        _scatter_body,
        grid_spec=grid_spec,
        out_shape=[
            jax.ShapeDtypeStruct((DATA_SIZE, 1, FEATURE_DIM), jnp.float32),
            jax.ShapeDtypeStruct((DATA_SIZE, 1, FEATURE_DIM), jnp.float32),
        ],
        input_output_aliases={3: 0, 4: 1},
        compiler_params=pltpu.CompilerParams(
            dimension_semantics=("arbitrary",),
        ),
    )(y, upd_f3, upd_p3, fringe3, phase3)
    return (nf3.reshape(DATA_SIZE, FEATURE_DIM),
            np3.reshape(DATA_SIZE, FEATURE_DIM))


def _sc_gather(idx, sf, sp):
    mesh = plsc.VectorSubcoreMesh(core_axis_name="c", subcore_axis_name="s")
    info = plsc.get_sparse_core_info()
    n_cores = info.num_cores

    @functools.partial(
        pl.kernel,
        mesh=mesh,
        out_type=[
            jax.ShapeDtypeStruct((BATCH, DATA_SIZE), jnp.float32),
            jax.ShapeDtypeStruct((BATCH, DATA_SIZE), jnp.float32),
        ],
        scratch_types=[
            pltpu.VMEM((DATA_SIZE,), jnp.int32),
            pltpu.VMEM((DATA_SIZE,), jnp.float32),
            pltpu.VMEM((DATA_SIZE,), jnp.float32),
            pltpu.VMEM((DATA_SIZE,), jnp.float32),
            pltpu.VMEM((DATA_SIZE,), jnp.float32),
        ],
        compiler_params=pltpu.CompilerParams(needs_layout_passes=False),
    )
    def k(idx_hbm, sf_hbm, sp_hbm, outp_hbm, outf_hbm,
          idx_v, sf_v, sp_v, outp_v, outf_v):
        wid = lax.axis_index("s") * n_cores + lax.axis_index("c")
        pltpu.sync_copy(idx_hbm.at[wid], idx_v)
        pltpu.sync_copy(sf_hbm.at[wid], sf_v)
        pltpu.sync_copy(sp_hbm.at[wid], sp_v)

        def body(j, carry):
            base = j * (LANES * UNROLL)
            for u in range(UNROLL):
                o = base + u * LANES
                v_idx = idx_v[pl.ds(o, LANES)]
                outp_v[pl.ds(o, LANES)] = plsc.load_gather(sf_v, [v_idx])
                outf_v[pl.ds(o, LANES)] = plsc.load_gather(sp_v, [v_idx])
            return carry

        lax.fori_loop(0, DATA_SIZE // (LANES * UNROLL), body, 0)
        pltpu.sync_copy(outp_v, outp_hbm.at[wid])
        pltpu.sync_copy(outf_v, outf_hbm.at[wid])

    return k(idx, sf, sp)


def kernel(fea_f, fea_p, y, idx, memory_fringe, memory_phase):
    y32 = y.astype(jnp.int32)
    yv = y32.reshape(BATCH, 1)
    idx32 = idx.astype(jnp.int32)
    sf, sp, upd_f, upd_p = _scores_call(
        memory_fringe, memory_phase, fea_f, fea_p, yv)
    new_fringe, new_phase = _scatter_call(
        y32, upd_f, upd_p, memory_fringe, memory_phase)
    out_phase, out_fringe = _sc_gather(idx32, sf, sp)
    return (out_fringe.reshape(BATCH, DATA_SIZE, 1),
            out_phase.reshape(BATCH, DATA_SIZE, 1),
            new_fringe, new_phase)


# speedup vs baseline: 1.1713x; 1.1713x over previous
import functools

import jax
import jax.numpy as jnp
from jax import lax
from jax.experimental import pallas as pl
from jax.experimental.pallas import tpu as pltpu
from jax.experimental.pallas import tpu_sc as plsc

FEATURE_DIM = 128
DATA_SIZE = 16384
BATCH = 32
T = 0.07
MOMENTUM = 0.5

ROW_BLK = 2048
NUM_BLKS = DATA_SIZE // ROW_BLK
LANES = 16
UNROLL = 4


def _tc_body(fringe_ref, phase_ref, fea_f_ref, fea_p_ref, y_ref,
             sf_ref, sp_ref, nf_ref, np_ref):
    i = pl.program_id(0)
    fringe = fringe_ref[...]
    phase = phase_ref[...]
    ff = fea_f_ref[...]
    fp = fea_p_ref[...]
    inv_t = jnp.float32(1.0 / T)

    dn_t = (((1,), (1,)), ((), ()))
    sf_ref[...] = lax.dot_general(fp, fringe, dn_t,
                                  preferred_element_type=jnp.float32) * inv_t
    sp_ref[...] = lax.dot_general(ff, phase, dn_t,
                                  preferred_element_type=jnp.float32) * inv_t

    y = y_ref[...]
    r = y - i * ROW_BLK
    col = lax.broadcasted_iota(jnp.int32, (BATCH, ROW_BLK), 1)
    onehot = (col == r).astype(jnp.float32)

    yrow = jnp.reshape(y, (1, BATCH))
    eq = y == yrow
    later = (lax.broadcasted_iota(jnp.int32, (BATCH, BATCH), 1)
             > lax.broadcasted_iota(jnp.int32, (BATCH, BATCH), 0))
    dup_later = jnp.any(eq & later, axis=1, keepdims=True)
    oh = onehot * jnp.where(dup_later, 0.0, 1.0)

    dn_g = (((1,), (0,)), ((), ()))
    gf = lax.dot_general(oh, fringe, dn_g, preferred_element_type=jnp.float32)
    gp = lax.dot_general(oh, phase, dn_g, preferred_element_type=jnp.float32)
    uf = MOMENTUM * gf + (1.0 - MOMENTUM) * ff
    up = MOMENTUM * gp + (1.0 - MOMENTUM) * fp
    unf = uf / jnp.maximum(jnp.sqrt(jnp.sum(uf * uf, axis=1, keepdims=True)),
                           1e-30)
    unp = up / jnp.maximum(jnp.sqrt(jnp.sum(up * up, axis=1, keepdims=True)),
                           1e-30)

    dn_s = (((0,), (0,)), ((), ()))
    scat_f = lax.dot_general(oh, unf, dn_s, preferred_element_type=jnp.float32)
    scat_p = lax.dot_general(oh, unp, dn_s, preferred_element_type=jnp.float32)
    ones = jnp.ones((BATCH, FEATURE_DIM), jnp.float32)
    rowcnt = lax.dot_general(oh, ones, dn_s,
                             preferred_element_type=jnp.float32)
    nf_ref[...] = fringe * (1.0 - rowcnt) + scat_f
    np_ref[...] = phase * (1.0 - rowcnt) + scat_p


def _tc_call(fringe, phase, ff, fp, yv):
    return pl.pallas_call(
        _tc_body,
        grid=(NUM_BLKS,),
        in_specs=[
            pl.BlockSpec((ROW_BLK, FEATURE_DIM), lambda i: (i, 0)),
            pl.BlockSpec((ROW_BLK, FEATURE_DIM), lambda i: (i, 0)),
            pl.BlockSpec((BATCH, FEATURE_DIM), lambda i: (0, 0)),
            pl.BlockSpec((BATCH, FEATURE_DIM), lambda i: (0, 0)),
            pl.BlockSpec((BATCH, 1), lambda i: (0, 0)),
        ],
        out_specs=[
            pl.BlockSpec((BATCH, ROW_BLK), lambda i: (0, i)),
            pl.BlockSpec((BATCH, ROW_BLK), lambda i: (0, i)),
            pl.BlockSpec((ROW_BLK, FEATURE_DIM), lambda i: (i, 0)),
            pl.BlockSpec((ROW_BLK, FEATURE_DIM), lambda i: (i, 0)),
        ],
        out_shape=[
            jax.ShapeDtypeStruct((BATCH, DATA_SIZE), jnp.float32),
            jax.ShapeDtypeStruct((BATCH, DATA_SIZE), jnp.float32),
            jax.ShapeDtypeStruct((DATA_SIZE, FEATURE_DIM), jnp.float32),
            jax.ShapeDtypeStruct((DATA_SIZE, FEATURE_DIM), jnp.float32),
        ],
        compiler_params=pltpu.CompilerParams(
            dimension_semantics=("arbitrary",),
        ),
    )(fringe, phase, ff, fp, yv)


def _sc_gather(idx, sf, sp):
    mesh = plsc.VectorSubcoreMesh(core_axis_name="c", subcore_axis_name="s")
    info = plsc.get_sparse_core_info()
    n_cores = info.num_cores

    @functools.partial(
        pl.kernel,
        mesh=mesh,
        out_type=[
            jax.ShapeDtypeStruct((BATCH, DATA_SIZE), jnp.float32),
            jax.ShapeDtypeStruct((BATCH, DATA_SIZE), jnp.float32),
        ],
        scratch_types=[
            pltpu.VMEM((DATA_SIZE,), jnp.int32),
            pltpu.VMEM((DATA_SIZE,), jnp.float32),
            pltpu.VMEM((DATA_SIZE,), jnp.float32),
            pltpu.VMEM((DATA_SIZE,), jnp.float32),
            pltpu.VMEM((DATA_SIZE,), jnp.float32),
        ],
        compiler_params=pltpu.CompilerParams(needs_layout_passes=False),
    )
    def k(idx_hbm, sf_hbm, sp_hbm, outp_hbm, outf_hbm,
          idx_v, sf_v, sp_v, outp_v, outf_v):
        wid = lax.axis_index("s") * n_cores + lax.axis_index("c")
        pltpu.sync_copy(idx_hbm.at[wid], idx_v)
        pltpu.sync_copy(sf_hbm.at[wid], sf_v)
        pltpu.sync_copy(sp_hbm.at[wid], sp_v)

        def body(j, carry):
            base = j * (LANES * UNROLL)
            for u in range(UNROLL):
                o = base + u * LANES
                v_idx = idx_v[pl.ds(o, LANES)]
                outp_v[pl.ds(o, LANES)] = plsc.load_gather(sf_v, [v_idx])
                outf_v[pl.ds(o, LANES)] = plsc.load_gather(sp_v, [v_idx])
            return carry

        lax.fori_loop(0, DATA_SIZE // (LANES * UNROLL), body, 0)
        pltpu.sync_copy(outp_v, outp_hbm.at[wid])
        pltpu.sync_copy(outf_v, outf_hbm.at[wid])

    return k(idx, sf, sp)


def kernel(fea_f, fea_p, y, idx, memory_fringe, memory_phase):
    yv = y.astype(jnp.int32).reshape(BATCH, 1)
    idx32 = idx.astype(jnp.int32)
    sf, sp, new_fringe, new_phase = _tc_call(
        memory_fringe, memory_phase, fea_f, fea_p, yv)
    out_phase, out_fringe = _sc_gather(idx32, sf, sp)
    return (out_fringe.reshape(BATCH, DATA_SIZE, 1),
            out_phase.reshape(BATCH, DATA_SIZE, 1),
            new_fringe, new_phase)


# speedup vs baseline: 1.2169x; 1.0390x over previous
import functools

import jax
import jax.numpy as jnp
from jax import lax
from jax.experimental import pallas as pl
from jax.experimental.pallas import tpu as pltpu
from jax.experimental.pallas import tpu_sc as plsc

FEATURE_DIM = 128
DATA_SIZE = 16384
BATCH = 32
T = 0.07
MOMENTUM = 0.5

ROW_BLK = 4096
NUM_BLKS = DATA_SIZE // ROW_BLK
LANES = 16
UNROLL = 4


def _tc_body(fringe_ref, phase_ref, fea_f_ref, fea_p_ref, y_ref,
             sf_ref, sp_ref, nf_ref, np_ref):
    i = pl.program_id(0)
    fringe = fringe_ref[...]
    phase = phase_ref[...]
    ff = fea_f_ref[...]
    fp = fea_p_ref[...]
    inv_t = jnp.float32(1.0 / T)

    dn_t = (((1,), (1,)), ((), ()))
    sf_ref[...] = lax.dot_general(fp, fringe, dn_t,
                                  preferred_element_type=jnp.float32) * inv_t
    sp_ref[...] = lax.dot_general(ff, phase, dn_t,
                                  preferred_element_type=jnp.float32) * inv_t

    y = y_ref[...]
    r = y - i * ROW_BLK
    col = lax.broadcasted_iota(jnp.int32, (BATCH, ROW_BLK), 1)
    onehot = (col == r).astype(jnp.float32)

    yrow = jnp.reshape(y, (1, BATCH))
    eq = y == yrow
    later = (lax.broadcasted_iota(jnp.int32, (BATCH, BATCH), 1)
             > lax.broadcasted_iota(jnp.int32, (BATCH, BATCH), 0))
    dup_later = jnp.any(eq & later, axis=1, keepdims=True)
    oh = onehot * jnp.where(dup_later, 0.0, 1.0)

    dn_g = (((1,), (0,)), ((), ()))
    gf = lax.dot_general(oh, fringe, dn_g, preferred_element_type=jnp.float32)
    gp = lax.dot_general(oh, phase, dn_g, preferred_element_type=jnp.float32)
    uf = MOMENTUM * gf + (1.0 - MOMENTUM) * ff
    up = MOMENTUM * gp + (1.0 - MOMENTUM) * fp
    unf = uf / jnp.maximum(jnp.sqrt(jnp.sum(uf * uf, axis=1, keepdims=True)),
                           1e-30)
    unp = up / jnp.maximum(jnp.sqrt(jnp.sum(up * up, axis=1, keepdims=True)),
                           1e-30)

    dn_s = (((0,), (0,)), ((), ()))
    scat_f = lax.dot_general(oh, unf, dn_s, preferred_element_type=jnp.float32)
    scat_p = lax.dot_general(oh, unp, dn_s, preferred_element_type=jnp.float32)
    ones = jnp.ones((BATCH, FEATURE_DIM), jnp.float32)
    rowcnt = lax.dot_general(oh, ones, dn_s,
                             preferred_element_type=jnp.float32)
    nf_ref[...] = fringe * (1.0 - rowcnt) + scat_f
    np_ref[...] = phase * (1.0 - rowcnt) + scat_p


def _tc_call(fringe, phase, ff, fp, yv):
    return pl.pallas_call(
        _tc_body,
        grid=(NUM_BLKS,),
        in_specs=[
            pl.BlockSpec((ROW_BLK, FEATURE_DIM), lambda i: (i, 0)),
            pl.BlockSpec((ROW_BLK, FEATURE_DIM), lambda i: (i, 0)),
            pl.BlockSpec((BATCH, FEATURE_DIM), lambda i: (0, 0)),
            pl.BlockSpec((BATCH, FEATURE_DIM), lambda i: (0, 0)),
            pl.BlockSpec((BATCH, 1), lambda i: (0, 0)),
        ],
        out_specs=[
            pl.BlockSpec((BATCH, ROW_BLK), lambda i: (0, i)),
            pl.BlockSpec((BATCH, ROW_BLK), lambda i: (0, i)),
            pl.BlockSpec((ROW_BLK, FEATURE_DIM), lambda i: (i, 0)),
            pl.BlockSpec((ROW_BLK, FEATURE_DIM), lambda i: (i, 0)),
        ],
        out_shape=[
            jax.ShapeDtypeStruct((BATCH, DATA_SIZE), jnp.float32),
            jax.ShapeDtypeStruct((BATCH, DATA_SIZE), jnp.float32),
            jax.ShapeDtypeStruct((DATA_SIZE, FEATURE_DIM), jnp.float32),
            jax.ShapeDtypeStruct((DATA_SIZE, FEATURE_DIM), jnp.float32),
        ],
        compiler_params=pltpu.CompilerParams(
            dimension_semantics=("arbitrary",),
        ),
    )(fringe, phase, ff, fp, yv)


def _sc_gather(idx, sf, sp):
    mesh = plsc.VectorSubcoreMesh(core_axis_name="c", subcore_axis_name="s")
    info = plsc.get_sparse_core_info()
    n_cores = info.num_cores

    @functools.partial(
        pl.kernel,
        mesh=mesh,
        out_type=[
            jax.ShapeDtypeStruct((BATCH, DATA_SIZE), jnp.float32),
            jax.ShapeDtypeStruct((BATCH, DATA_SIZE), jnp.float32),
        ],
        scratch_types=[
            pltpu.VMEM((DATA_SIZE,), jnp.int32),
            pltpu.VMEM((DATA_SIZE,), jnp.float32),
            pltpu.VMEM((DATA_SIZE,), jnp.float32),
            pltpu.VMEM((DATA_SIZE,), jnp.float32),
            pltpu.VMEM((DATA_SIZE,), jnp.float32),
        ],
        compiler_params=pltpu.CompilerParams(needs_layout_passes=False),
    )
    def k(idx_hbm, sf_hbm, sp_hbm, outp_hbm, outf_hbm,
          idx_v, sf_v, sp_v, outp_v, outf_v):
        wid = lax.axis_index("s") * n_cores + lax.axis_index("c")
        pltpu.sync_copy(idx_hbm.at[wid], idx_v)
        pltpu.sync_copy(sf_hbm.at[wid], sf_v)
        pltpu.sync_copy(sp_hbm.at[wid], sp_v)

        def body(j, carry):
            base = j * (LANES * UNROLL)
            for u in range(UNROLL):
                o = base + u * LANES
                v_idx = idx_v[pl.ds(o, LANES)]
                outp_v[pl.ds(o, LANES)] = plsc.load_gather(sf_v, [v_idx])
                outf_v[pl.ds(o, LANES)] = plsc.load_gather(sp_v, [v_idx])
            return carry

        lax.fori_loop(0, DATA_SIZE // (LANES * UNROLL), body, 0)
        pltpu.sync_copy(outp_v, outp_hbm.at[wid])
        pltpu.sync_copy(outf_v, outf_hbm.at[wid])

    return k(idx, sf, sp)


def kernel(fea_f, fea_p, y, idx, memory_fringe, memory_phase):
    yv = y.astype(jnp.int32).reshape(BATCH, 1)
    idx32 = idx.astype(jnp.int32)
    sf, sp, new_fringe, new_phase = _tc_call(
        memory_fringe, memory_phase, fea_f, fea_p, yv)
    out_phase, out_fringe = _sc_gather(idx32, sf, sp)
    return (out_fringe.reshape(BATCH, DATA_SIZE, 1),
            out_phase.reshape(BATCH, DATA_SIZE, 1),
            new_fringe, new_phase)


# speedup vs baseline: 1.2319x; 1.0123x over previous
import functools

import jax
import jax.numpy as jnp
from jax import lax
from jax.experimental import pallas as pl
from jax.experimental.pallas import tpu as pltpu
from jax.experimental.pallas import tpu_sc as plsc

FEATURE_DIM = 128
DATA_SIZE = 16384
BATCH = 32
T = 0.07
MOMENTUM = 0.5

ROW_BLK = 8192
NUM_BLKS = DATA_SIZE // ROW_BLK
LANES = 16
UNROLL = 4


def _tc_body(fringe_ref, phase_ref, fea_f_ref, fea_p_ref, y_ref,
             sf_ref, sp_ref, nf_ref, np_ref):
    i = pl.program_id(0)
    fringe = fringe_ref[...]
    phase = phase_ref[...]
    ff = fea_f_ref[...]
    fp = fea_p_ref[...]
    inv_t = jnp.float32(1.0 / T)

    dn_t = (((1,), (1,)), ((), ()))
    sf_ref[...] = lax.dot_general(fp, fringe, dn_t,
                                  preferred_element_type=jnp.float32) * inv_t
    sp_ref[...] = lax.dot_general(ff, phase, dn_t,
                                  preferred_element_type=jnp.float32) * inv_t

    y = y_ref[...]
    r = y - i * ROW_BLK
    col = lax.broadcasted_iota(jnp.int32, (BATCH, ROW_BLK), 1)
    onehot = (col == r).astype(jnp.float32)

    yrow = jnp.reshape(y, (1, BATCH))
    eq = y == yrow
    later = (lax.broadcasted_iota(jnp.int32, (BATCH, BATCH), 1)
             > lax.broadcasted_iota(jnp.int32, (BATCH, BATCH), 0))
    dup_later = jnp.any(eq & later, axis=1, keepdims=True)
    oh = onehot * jnp.where(dup_later, 0.0, 1.0)

    dn_g = (((1,), (0,)), ((), ()))
    gf = lax.dot_general(oh, fringe, dn_g, preferred_element_type=jnp.float32)
    gp = lax.dot_general(oh, phase, dn_g, preferred_element_type=jnp.float32)
    uf = MOMENTUM * gf + (1.0 - MOMENTUM) * ff
    up = MOMENTUM * gp + (1.0 - MOMENTUM) * fp
    unf = uf / jnp.maximum(jnp.sqrt(jnp.sum(uf * uf, axis=1, keepdims=True)),
                           1e-30)
    unp = up / jnp.maximum(jnp.sqrt(jnp.sum(up * up, axis=1, keepdims=True)),
                           1e-30)

    dn_s = (((0,), (0,)), ((), ()))
    scat_f = lax.dot_general(oh, unf, dn_s, preferred_element_type=jnp.float32)
    scat_p = lax.dot_general(oh, unp, dn_s, preferred_element_type=jnp.float32)
    ones = jnp.ones((BATCH, FEATURE_DIM), jnp.float32)
    rowcnt = lax.dot_general(oh, ones, dn_s,
                             preferred_element_type=jnp.float32)
    nf_ref[...] = fringe * (1.0 - rowcnt) + scat_f
    np_ref[...] = phase * (1.0 - rowcnt) + scat_p


def _tc_call(fringe, phase, ff, fp, yv):
    return pl.pallas_call(
        _tc_body,
        grid=(NUM_BLKS,),
        in_specs=[
            pl.BlockSpec((ROW_BLK, FEATURE_DIM), lambda i: (i, 0)),
            pl.BlockSpec((ROW_BLK, FEATURE_DIM), lambda i: (i, 0)),
            pl.BlockSpec((BATCH, FEATURE_DIM), lambda i: (0, 0)),
            pl.BlockSpec((BATCH, FEATURE_DIM), lambda i: (0, 0)),
            pl.BlockSpec((BATCH, 1), lambda i: (0, 0)),
        ],
        out_specs=[
            pl.BlockSpec((BATCH, ROW_BLK), lambda i: (0, i)),
            pl.BlockSpec((BATCH, ROW_BLK), lambda i: (0, i)),
            pl.BlockSpec((ROW_BLK, FEATURE_DIM), lambda i: (i, 0)),
            pl.BlockSpec((ROW_BLK, FEATURE_DIM), lambda i: (i, 0)),
        ],
        out_shape=[
            jax.ShapeDtypeStruct((BATCH, DATA_SIZE), jnp.float32),
            jax.ShapeDtypeStruct((BATCH, DATA_SIZE), jnp.float32),
            jax.ShapeDtypeStruct((DATA_SIZE, FEATURE_DIM), jnp.float32),
            jax.ShapeDtypeStruct((DATA_SIZE, FEATURE_DIM), jnp.float32),
        ],
        compiler_params=pltpu.CompilerParams(
            dimension_semantics=("arbitrary",),
        ),
    )(fringe, phase, ff, fp, yv)


def _sc_gather(idx, sf, sp):
    mesh = plsc.VectorSubcoreMesh(core_axis_name="c", subcore_axis_name="s")
    info = plsc.get_sparse_core_info()
    n_cores = info.num_cores

    @functools.partial(
        pl.kernel,
        mesh=mesh,
        out_type=[
            jax.ShapeDtypeStruct((BATCH, DATA_SIZE), jnp.float32),
            jax.ShapeDtypeStruct((BATCH, DATA_SIZE), jnp.float32),
        ],
        scratch_types=[
            pltpu.VMEM((DATA_SIZE,), jnp.int32),
            pltpu.VMEM((DATA_SIZE,), jnp.float32),
            pltpu.VMEM((DATA_SIZE,), jnp.float32),
            pltpu.VMEM((DATA_SIZE,), jnp.float32),
            pltpu.VMEM((DATA_SIZE,), jnp.float32),
        ],
        compiler_params=pltpu.CompilerParams(needs_layout_passes=False),
    )
    def k(idx_hbm, sf_hbm, sp_hbm, outp_hbm, outf_hbm,
          idx_v, sf_v, sp_v, outp_v, outf_v):
        wid = lax.axis_index("s") * n_cores + lax.axis_index("c")
        pltpu.sync_copy(idx_hbm.at[wid], idx_v)
        pltpu.sync_copy(sf_hbm.at[wid], sf_v)
        pltpu.sync_copy(sp_hbm.at[wid], sp_v)

        def body(j, carry):
            base = j * (LANES * UNROLL)
            for u in range(UNROLL):
                o = base + u * LANES
                v_idx = idx_v[pl.ds(o, LANES)]
                outp_v[pl.ds(o, LANES)] = plsc.load_gather(sf_v, [v_idx])
                outf_v[pl.ds(o, LANES)] = plsc.load_gather(sp_v, [v_idx])
            return carry

        lax.fori_loop(0, DATA_SIZE // (LANES * UNROLL), body, 0)
        pltpu.sync_copy(outp_v, outp_hbm.at[wid])
        pltpu.sync_copy(outf_v, outf_hbm.at[wid])

    return k(idx, sf, sp)


def kernel(fea_f, fea_p, y, idx, memory_fringe, memory_phase):
    yv = y.astype(jnp.int32).reshape(BATCH, 1)
    idx32 = idx.astype(jnp.int32)
    sf, sp, new_fringe, new_phase = _tc_call(
        memory_fringe, memory_phase, fea_f, fea_p, yv)
    out_phase, out_fringe = _sc_gather(idx32, sf, sp)
    return (out_fringe.reshape(BATCH, DATA_SIZE, 1),
            out_phase.reshape(BATCH, DATA_SIZE, 1),
            new_fringe, new_phase)


# speedup vs baseline: 1.2848x; 1.0429x over previous
import functools

import jax
import jax.numpy as jnp
from jax import lax
from jax.experimental import pallas as pl
from jax.experimental.pallas import tpu as pltpu
from jax.experimental.pallas import tpu_sc as plsc

FEATURE_DIM = 128
DATA_SIZE = 16384
BATCH = 32
T = 0.07
MOMENTUM = 0.5

ROW_BLK = 8192
NUM_BLKS = DATA_SIZE // ROW_BLK
LANES = 16
UNROLL = 4


def _scores_body(fringe_ref, phase_ref, fea_f_ref, fea_p_ref, y_ref,
                 sf_ref, sp_ref, uf_ref, up_ref):
    i = pl.program_id(0)
    fringe = fringe_ref[...]
    phase = phase_ref[...]
    ff = fea_f_ref[...]
    fp = fea_p_ref[...]
    inv_t = jnp.float32(1.0 / T)

    dn_t = (((1,), (1,)), ((), ()))
    sf_ref[...] = lax.dot_general(fp, fringe, dn_t,
                                  preferred_element_type=jnp.float32) * inv_t
    sp_ref[...] = lax.dot_general(ff, phase, dn_t,
                                  preferred_element_type=jnp.float32) * inv_t

    y = y_ref[...]
    r = y - i * ROW_BLK
    col = lax.broadcasted_iota(jnp.int32, (BATCH, ROW_BLK), 1)
    onehot = (col == r).astype(jnp.float32)

    @pl.when(i == 0)
    def _():
        uf_ref[...] = (1.0 - MOMENTUM) * ff
        up_ref[...] = (1.0 - MOMENTUM) * fp

    dn_g = (((1,), (0,)), ((), ()))
    uf_ref[...] += MOMENTUM * lax.dot_general(
        onehot, fringe, dn_g, preferred_element_type=jnp.float32)
    up_ref[...] += MOMENTUM * lax.dot_general(
        onehot, phase, dn_g, preferred_element_type=jnp.float32)

    @pl.when(i == NUM_BLKS - 1)
    def _():
        uf = uf_ref[...]
        up = up_ref[...]
        uf_ref[...] = uf / jnp.maximum(
            jnp.sqrt(jnp.sum(uf * uf, axis=1, keepdims=True)), 1e-30)
        up_ref[...] = up / jnp.maximum(
            jnp.sqrt(jnp.sum(up * up, axis=1, keepdims=True)), 1e-30)


def _scores_call(fringe, phase, ff, fp, yv):
    return pl.pallas_call(
        _scores_body,
        grid=(NUM_BLKS,),
        in_specs=[
            pl.BlockSpec((ROW_BLK, FEATURE_DIM), lambda i: (i, 0)),
            pl.BlockSpec((ROW_BLK, FEATURE_DIM), lambda i: (i, 0)),
            pl.BlockSpec((BATCH, FEATURE_DIM), lambda i: (0, 0)),
            pl.BlockSpec((BATCH, FEATURE_DIM), lambda i: (0, 0)),
            pl.BlockSpec((BATCH, 1), lambda i: (0, 0)),
        ],
        out_specs=[
            pl.BlockSpec((BATCH, ROW_BLK), lambda i: (0, i)),
            pl.BlockSpec((BATCH, ROW_BLK), lambda i: (0, i)),
            pl.BlockSpec((BATCH, FEATURE_DIM), lambda i: (0, 0)),
            pl.BlockSpec((BATCH, FEATURE_DIM), lambda i: (0, 0)),
        ],
        out_shape=[
            jax.ShapeDtypeStruct((BATCH, DATA_SIZE), jnp.float32),
            jax.ShapeDtypeStruct((BATCH, DATA_SIZE), jnp.float32),
            jax.ShapeDtypeStruct((BATCH, FEATURE_DIM), jnp.float32),
            jax.ShapeDtypeStruct((BATCH, FEATURE_DIM), jnp.float32),
        ],
        compiler_params=pltpu.CompilerParams(
            dimension_semantics=("arbitrary",),
        ),
    )(fringe, phase, ff, fp, yv)


def _update_body(fringe_ref, phase_ref, unf_ref, unp_ref, y_ref,
                 nf_ref, np_ref):
    i = pl.program_id(0)
    fringe = fringe_ref[...]
    phase = phase_ref[...]
    unf = unf_ref[...]
    unp = unp_ref[...]
    y = y_ref[...]
    r = y - i * ROW_BLK
    col = lax.broadcasted_iota(jnp.int32, (BATCH, ROW_BLK), 1)
    onehot = (col == r).astype(jnp.float32)

    yrow = jnp.reshape(y, (1, BATCH))
    eq = y == yrow
    later = (lax.broadcasted_iota(jnp.int32, (BATCH, BATCH), 1)
             > lax.broadcasted_iota(jnp.int32, (BATCH, BATCH), 0))
    dup_later = jnp.any(eq & later, axis=1, keepdims=True)
    oh = onehot * jnp.where(dup_later, 0.0, 1.0)

    dn_s = (((0,), (0,)), ((), ()))
    scat_f = lax.dot_general(oh, unf, dn_s, preferred_element_type=jnp.float32)
    scat_p = lax.dot_general(oh, unp, dn_s, preferred_element_type=jnp.float32)
    ones = jnp.ones((BATCH, FEATURE_DIM), jnp.float32)
    rowcnt = lax.dot_general(oh, ones, dn_s,
                             preferred_element_type=jnp.float32)
    nf_ref[...] = fringe * (1.0 - rowcnt) + scat_f
    np_ref[...] = phase * (1.0 - rowcnt) + scat_p


def _update_call(fringe, phase, unf, unp, yv):
    return pl.pallas_call(
        _update_body,
        grid=(NUM_BLKS,),
        in_specs=[
            pl.BlockSpec((ROW_BLK, FEATURE_DIM), lambda i: (i, 0)),
            pl.BlockSpec((ROW_BLK, FEATURE_DIM), lambda i: (i, 0)),
            pl.BlockSpec((BATCH, FEATURE_DIM), lambda i: (0, 0)),
            pl.BlockSpec((BATCH, FEATURE_DIM), lambda i: (0, 0)),
            pl.BlockSpec((BATCH, 1), lambda i: (0, 0)),
        ],
        out_specs=[
            pl.BlockSpec((ROW_BLK, FEATURE_DIM), lambda i: (i, 0)),
            pl.BlockSpec((ROW_BLK, FEATURE_DIM), lambda i: (i, 0)),
        ],
        out_shape=[
            jax.ShapeDtypeStruct((DATA_SIZE, FEATURE_DIM), jnp.float32),
            jax.ShapeDtypeStruct((DATA_SIZE, FEATURE_DIM), jnp.float32),
        ],
        compiler_params=pltpu.CompilerParams(
            dimension_semantics=("arbitrary",),
        ),
    )(fringe, phase, unf, unp, yv)


def _sc_gather(idx, sf, sp):
    mesh = plsc.VectorSubcoreMesh(core_axis_name="c", subcore_axis_name="s")
    info = plsc.get_sparse_core_info()
    n_cores = info.num_cores

    @functools.partial(
        pl.kernel,
        mesh=mesh,
        out_type=[
            jax.ShapeDtypeStruct((BATCH, DATA_SIZE), jnp.float32),
            jax.ShapeDtypeStruct((BATCH, DATA_SIZE), jnp.float32),
        ],
        scratch_types=[
            pltpu.VMEM((DATA_SIZE,), jnp.int32),
            pltpu.VMEM((DATA_SIZE,), jnp.float32),
            pltpu.VMEM((DATA_SIZE,), jnp.float32),
            pltpu.VMEM((DATA_SIZE,), jnp.float32),
            pltpu.VMEM((DATA_SIZE,), jnp.float32),
        ],
        compiler_params=pltpu.CompilerParams(needs_layout_passes=False),
    )
    def k(idx_hbm, sf_hbm, sp_hbm, outp_hbm, outf_hbm,
          idx_v, sf_v, sp_v, outp_v, outf_v):
        wid = lax.axis_index("s") * n_cores + lax.axis_index("c")
        pltpu.sync_copy(idx_hbm.at[wid], idx_v)
        pltpu.sync_copy(sf_hbm.at[wid], sf_v)
        pltpu.sync_copy(sp_hbm.at[wid], sp_v)

        def body(j, carry):
            base = j * (LANES * UNROLL)
            for u in range(UNROLL):
                o = base + u * LANES
                v_idx = idx_v[pl.ds(o, LANES)]
                outp_v[pl.ds(o, LANES)] = plsc.load_gather(sf_v, [v_idx])
                outf_v[pl.ds(o, LANES)] = plsc.load_gather(sp_v, [v_idx])
            return carry

        lax.fori_loop(0, DATA_SIZE // (LANES * UNROLL), body, 0)
        pltpu.sync_copy(outp_v, outp_hbm.at[wid])
        pltpu.sync_copy(outf_v, outf_hbm.at[wid])

    return k(idx, sf, sp)


def kernel(fea_f, fea_p, y, idx, memory_fringe, memory_phase):
    yv = y.astype(jnp.int32).reshape(BATCH, 1)
    idx32 = idx.astype(jnp.int32)
    sf, sp, unf, unp = _scores_call(
        memory_fringe, memory_phase, fea_f, fea_p, yv)
    out_phase, out_fringe = _sc_gather(idx32, sf, sp)
    new_fringe, new_phase = _update_call(
        memory_fringe, memory_phase, unf, unp, yv)
    return (out_fringe.reshape(BATCH, DATA_SIZE, 1),
            out_phase.reshape(BATCH, DATA_SIZE, 1),
            new_fringe, new_phase)
